# baseline (speedup 1.0000x reference)
import functools

import jax
import jax.numpy as jnp
from jax import lax
from jax.experimental import pallas as pl
from jax.experimental.pallas import tpu as pltpu
from jax.experimental.pallas import tpu_sc as plsc

NUM_ENT = 1000000
NUM_REL = 1000
EMB_DIM = 64
BATCH = 16384

_INFO = plsc.get_sparse_core_info()
_NC = _INFO.num_cores
_NS = _INFO.num_subcores
_NW = _NC * _NS
_BPW = BATCH // _NW
_LANES = 16
_GCH = 128
_PH = 2
_RPP = _BPW // _PH


def _dist_mult_body(sub_h, rel_h, obj_h, ent_h, rele_h, bias_h, out_h,
                    sidx, ridx, oidx, s2, r2, o2,
                    h_v, r_v, t_v, b_v, out_v, sem):
    sid = lax.axis_index("s")
    wid = sid * _NC + lax.axis_index("c")
    base = wid * _BPW

    cb_s = pltpu.async_copy(sub_h.at[pl.ds(base, _BPW)], sidx, sem)
    cb_r = pltpu.async_copy(rel_h.at[pl.ds(base, _BPW)], ridx, sem)
    cb_o = pltpu.async_copy(obj_h.at[pl.ds(base, _BPW)], oidx, sem)
    cb_s.wait()
    cb_r.wait()
    cb_o.wait()

    for i in range(_BPW // _LANES):
        sl = pl.ds(i * _LANES, _LANES)
        s2[sl] = jnp.right_shift(sidx[sl], 1)
        r2[sl] = jnp.right_shift(ridx[sl], 1)
        o2[sl] = jnp.right_shift(oidx[sl], 1)

    bias_cbs = [
        pltpu.async_copy(bias_h.at[oidx.at[pl.ds(j * _GCH, _GCH)]],
                         b_v.at[pl.ds(j * _GCH, _GCH)], sem)
        for j in range(_BPW // _GCH)
    ]

    lane = lax.iota(jnp.int32, _LANES)
    perms = [jnp.reshape((lane + sh) & (_LANES - 1), (_LANES, 1))
             for sh in (8, 4, 2, 1)]
    gdn = lax.GatherDimensionNumbers(offset_dims=(),
                                     collapsed_slice_dims=(0,),
                                     start_index_map=(0,))

    def shuffle(p, perm):
        return lax.gather(p, perm, gdn, (1,),
                          mode=lax.GatherScatterMode.PROMISE_IN_BOUNDS)

    for ph in range(_PH):
        pbase = ph * _RPP
        cbs = []
        for j in range(_RPP // _GCH):
            js = pl.ds(pbase + j * _GCH, _GCH)
            dst = pl.ds(j * _GCH, _GCH)
            cbs.append(pltpu.async_copy(ent_h.at[s2.at[js]],
                                        h_v.at[dst], sem))
            cbs.append(pltpu.async_copy(rele_h.at[r2.at[js]],
                                        r_v.at[dst], sem))
            cbs.append(pltpu.async_copy(ent_h.at[o2.at[js]],
                                        t_v.at[dst], sem))
        for cb in cbs:
            cb.wait()
        if ph == 0:
            for cb in bias_cbs:
                cb.wait()

        def rows(i, carry):
            cb = i * _LANES
            csl = pl.ds(pbase + cb, _LANES)
            osv = (sidx[csl] & 1) * EMB_DIM
            orv = (ridx[csl] & 1) * EMB_DIM
            oov = (oidx[csl] & 1) * EMB_DIM
            tot = jnp.zeros((_LANES,), jnp.float32)
            for u in range(_LANES):
                row = cb + u
                offs = osv[u]
                offr = orv[u]
                offo = oov[u]
                p = (h_v[row, pl.ds(offs, _LANES)]
                     * r_v[row, pl.ds(offr, _LANES)]
                     * t_v[row, pl.ds(offo, _LANES)])
                for q in range(1, EMB_DIM // _LANES):
                    p = p + (h_v[row, pl.ds(offs + q * _LANES, _LANES)]
                             * r_v[row, pl.ds(offr + q * _LANES, _LANES)]
                             * t_v[row, pl.ds(offo + q * _LANES, _LANES)])
                for perm in perms:
                    p = p + shuffle(p, perm)
                tot = jnp.where(lane == u, p, tot)
            sc = tot + b_v[pl.ds(pbase + cb, _LANES)]
            out_v[pl.ds(pbase + cb, _LANES)] = 1.0 / (1.0 + jnp.exp(-sc))
            return carry

        lax.fori_loop(0, _RPP // _LANES, rows, 0)

    pltpu.sync_copy(out_v, out_h.at[pl.ds(base, _BPW)])


@jax.jit
def kernel(sub, rel, obj, ent_emb, rel_emb, bias):
    mesh = plsc.VectorSubcoreMesh(core_axis_name="c", subcore_axis_name="s")
    k = functools.partial(
        pl.kernel,
        mesh=mesh,
        out_type=jax.ShapeDtypeStruct((BATCH,), jnp.float32),
        scratch_types=[
            pltpu.VMEM((_BPW,), jnp.int32),
            pltpu.VMEM((_BPW,), jnp.int32),
            pltpu.VMEM((_BPW,), jnp.int32),
            pltpu.VMEM((_BPW,), jnp.int32),
            pltpu.VMEM((_BPW,), jnp.int32),
            pltpu.VMEM((_BPW,), jnp.int32),
            pltpu.VMEM((_RPP, 2 * EMB_DIM), jnp.float32),
            pltpu.VMEM((_RPP, 2 * EMB_DIM), jnp.float32),
            pltpu.VMEM((_RPP, 2 * EMB_DIM), jnp.float32),
            pltpu.VMEM((_BPW,), jnp.float32),
            pltpu.VMEM((_BPW,), jnp.float32),
            pltpu.SemaphoreType.DMA,
        ],
    )(_dist_mult_body)
    return k(sub.astype(jnp.int32), rel.astype(jnp.int32),
             obj.astype(jnp.int32),
             ent_emb.reshape(NUM_ENT // 2, 2 * EMB_DIM),
             rel_emb.reshape(NUM_REL // 2, 2 * EMB_DIM),
             bias)
